# Initial kernel scaffold; baseline (speedup 1.0000x reference)
import jax
import jax.numpy as jnp
from jax.experimental import pallas as pl


def kernel(x, adj_0, adj_1, adj_2, Wq_0, Wk_0, Wm1_0, bm1_0, Wm2_0, bm2_0, Wq_1, Wk_1, Wm1_1, bm1_1, Wm2_1, bm2_1, Wq_2, Wk_2, Wm1_2, bm1_2, Wm2_2, bm2_2, Wout, bout, Wb1, bb1, Wb2, bb2, alpha, beta):
    raise NotImplementedError("write your pallas kernel here")


# speedup vs baseline: 6.4982x; 6.4982x over previous
import functools

import jax
import jax.numpy as jnp
from jax import lax
from jax.experimental import pallas as pl
from jax.experimental.pallas import tpu as pltpu
from jax.experimental.pallas import tpu_sc as plsc

N = 10000; D = 256; H = 8; PH = 32; ET = 3; E = 53334; MSG = H * PH; INTER = 1024
SMALL = 1e-7
E2 = 2 * E
E2P = 107520
E3 = ET * E2P
TBL = 2 * D + MSG
NW = 32
EPW = E3 // NW
GC = 32
NGC = EPW // GC
BE = 512
NBE = E3 // BE
NBT = E2P // BE
BN = 1000
SC = 32
NSC = E3 // 16 // SC
SCD = 32
NSCD = E3 // 2 // 16 // SCD
ZR = 32
DR = 640
N2 = N // 2
DND = 5008
DRD = 320


def _k1_body(x_ref, ws_ref, wt_ref, bs_ref, s_ref, t_ref):
    xb = x_ref[...]
    s_ref[0] = jnp.dot(xb, ws_ref[0], preferred_element_type=jnp.float32) + bs_ref[0]
    t_ref[0] = jnp.dot(xb, wt_ref[0], preferred_element_type=jnp.float32)


def _k1(x, WS, WT, bS):
    return pl.pallas_call(
        _k1_body,
        grid=(ET, N // BN),
        in_specs=[
            pl.BlockSpec((BN, D), lambda t, i: (i, 0)),
            pl.BlockSpec((1, D, TBL), lambda t, i: (t, 0, 0)),
            pl.BlockSpec((1, D, TBL), lambda t, i: (t, 0, 0)),
            pl.BlockSpec((1, 1, TBL), lambda t, i: (t, 0, 0)),
        ],
        out_specs=[
            pl.BlockSpec((1, BN, TBL), lambda t, i: (t, i, 0)),
            pl.BlockSpec((1, BN, TBL), lambda t, i: (t, i, 0)),
        ],
        out_shape=[
            jax.ShapeDtypeStruct((ET, N, TBL), jnp.float32),
            jax.ShapeDtypeStruct((ET, N, TBL), jnp.float32),
        ],
    )(x, WS, WT, bS)


def _k2_body(s_hbm, t_hbm, src_hbm, tgt_hbm, hpre_hbm, qk_hbm,
             sidx, tidx, srows, trows, hbuf, qbuf, sem_s, sem_t):
    cid = lax.axis_index("c")
    sid = lax.axis_index("s")
    wid = sid * 2 + cid
    base0 = wid * EPW

    def chunk(j, carry):
        base = base0 + j * GC
        pltpu.sync_copy(src_hbm.at[pl.ds(base, GC)], sidx)
        pltpu.sync_copy(tgt_hbm.at[pl.ds(base, GC)], tidx)
        cp1 = pltpu.async_copy(s_hbm.at[sidx], srows, sem_s)
        cp2 = pltpu.async_copy(t_hbm.at[tidx], trows, sem_t)
        cp1.wait()
        cp2.wait()

        def edge(e, c2):
            for i in range(2 * D // 16):
                sl = pl.ds(16 * i, 16)
                hbuf[e, sl] = srows[e, sl] + trows[e, sl]
            for i in range(MSG // 16):
                sl = pl.ds(2 * D + 16 * i, 16)
                so = pl.ds(16 * i, 16)
                qbuf[e, so] = srows[e, sl] * trows[e, sl]
            return c2

        lax.fori_loop(0, GC, edge, 0)
        pltpu.sync_copy(hbuf, hpre_hbm.at[pl.ds(base, GC)])
        pltpu.sync_copy(qbuf, qk_hbm.at[pl.ds(base, GC)])
        return carry

    lax.fori_loop(0, NGC, chunk, 0)


def _k2(Sflat, Tflat, src_all, tgt_all):
    mesh = plsc.VectorSubcoreMesh(core_axis_name="c", subcore_axis_name="s")
    fn = pl.kernel(
        _k2_body,
        out_type=[
            jax.ShapeDtypeStruct((E3, 2 * D), jnp.float32),
            jax.ShapeDtypeStruct((E3, MSG), jnp.float32),
        ],
        mesh=mesh,
        scratch_types=[
            pltpu.VMEM((GC,), jnp.int32),
            pltpu.VMEM((GC,), jnp.int32),
            pltpu.VMEM((GC, TBL), jnp.float32),
            pltpu.VMEM((GC, TBL), jnp.float32),
            pltpu.VMEM((GC, 2 * D), jnp.float32),
            pltpu.VMEM((GC, MSG), jnp.float32),
            pltpu.SemaphoreType.DMA,
            pltpu.SemaphoreType.DMA,
        ],
    )
    return fn(Sflat, Tflat, src_all, tgt_all)


def _k3_body(hpre_ref, qk_ref, wm2_ref, bm2_ref, wm_ref, ex_ref):
    h = jnp.maximum(hpre_ref[...], 0.0)
    m = jnp.dot(h, wm2_ref[0], preferred_element_type=jnp.float32) + bm2_ref[0]
    m = jnp.maximum(m, 0.0)
    qk = qk_ref[...]
    ci = lax.broadcasted_iota(jnp.int32, (MSG, H), 0) // PH
    hj = lax.broadcasted_iota(jnp.int32, (MSG, H), 1)
    BD = (ci == hj).astype(jnp.float32)
    s = jnp.dot(qk, BD, preferred_element_type=jnp.float32)
    gid = pl.program_id(0)
    lrow = (gid % NBT) * BE + lax.broadcasted_iota(jnp.int32, (BE, H), 0)
    ex = jnp.where(lrow < E2, jnp.exp(s), 0.0)
    hr = lax.broadcasted_iota(jnp.int32, (H, MSG), 0)
    cc = lax.broadcasted_iota(jnp.int32, (H, MSG), 1) // PH
    P8 = (hr == cc).astype(jnp.float32)
    wm = m * jnp.dot(ex, P8, preferred_element_type=jnp.float32)
    wm_ref[...] = wm
    er = lax.broadcasted_iota(jnp.int32, (H, 128), 0)
    ec = lax.broadcasted_iota(jnp.int32, (H, 128), 1)
    ex_ref[...] = jnp.dot(ex, (er == ec).astype(jnp.float32),
                          preferred_element_type=jnp.float32)


def _k3(hpre, qk, Wm2s, bm2s):
    return pl.pallas_call(
        _k3_body,
        grid=(NBE,),
        in_specs=[
            pl.BlockSpec((BE, 2 * D), lambda i: (i, 0)),
            pl.BlockSpec((BE, MSG), lambda i: (i, 0)),
            pl.BlockSpec((1, 2 * D, MSG), lambda i: (i // NBT, 0, 0)),
            pl.BlockSpec((1, 1, MSG), lambda i: (i // NBT, 0, 0)),
        ],
        out_specs=[
            pl.BlockSpec((BE, MSG), lambda i: (i, 0)),
            pl.BlockSpec((BE, 128), lambda i: (i, 0)),
        ],
        out_shape=[
            jax.ShapeDtypeStruct((E3, MSG), jnp.float32),
            jax.ShapeDtypeStruct((E3, 128), jnp.float32),
        ],
    )(hpre, qk, Wm2s, bm2s)


def _k4_body(wm_hbm, ex_hbm, tgt_hbm, tgt2_hbm, rows_hbm,
             agg_hbm, den_hbm,
             acc_agg, acc_den, zb, ridx, tidx, tidx2, sbuf):
    cid = lax.axis_index("c")
    sid = lax.axis_index("s")
    ccol = cid * (MSG // 2)

    def zrow(r, c2):
        for i in range(8):
            zb[r, pl.ds(16 * i, 16)] = jnp.zeros((16,), jnp.float32)
        return c2

    lax.fori_loop(0, ZR, zrow, 0)

    base_rows = jnp.minimum(sid * DR, N - DR)
    base_den = jnp.minimum(sid * DRD, DND - DRD)

    def stripe_idx(off):
        pltpu.sync_copy(rows_hbm.at[pl.ds(off, ZR)], ridx)

    def zcopy(b, c2):
        off = base_rows + b * ZR
        stripe_idx(off)
        pltpu.sync_copy(zb, acc_agg.at[ridx])
        return c2

    lax.fori_loop(0, DR // ZR, zcopy, 0)

    def zden(b, c2):
        off = base_den + b * ZR
        stripe_idx(off)
        pltpu.sync_copy(zb, acc_den.at[ridx])
        return c2

    lax.fori_loop(0, DRD // ZR, zden, 0)
    plsc.subcore_barrier()

    def sc_chunk(j, c2):
        base = sid * (E3 // 16) + j * SC
        pltpu.sync_copy(tgt_hbm.at[pl.ds(base, SC)], tidx)
        pltpu.sync_copy(wm_hbm.at[pl.ds(base, SC), pl.ds(ccol, MSG // 2)], sbuf)
        pltpu.sync_copy(sbuf, acc_agg.at[tidx], add=True)
        pltpu.sync_copy(tgt2_hbm.at[cid, pl.ds(base, SC)], tidx2)
        pltpu.sync_copy(ex_hbm.at[pl.ds(base, SC)], sbuf)
        pltpu.sync_copy(sbuf, acc_den.at[tidx2], add=True)
        return c2

    lax.fori_loop(0, NSC, sc_chunk, 0)
    plsc.subcore_barrier()

    def drain(b, c2):
        off = base_rows + b * ZR
        stripe_idx(off)
        pltpu.sync_copy(acc_agg.at[ridx], zb)
        pltpu.sync_copy(zb, agg_hbm.at[pl.ds(off, ZR), pl.ds(ccol, MSG // 2)])
        return c2

    lax.fori_loop(0, DR // ZR, drain, 0)

    base_dd = jnp.minimum(sid * DRD, N2 - DRD)

    def draind(b, c2):
        off = base_dd + b * ZR
        stripe_idx(off)
        pltpu.sync_copy(acc_den.at[ridx], zb)
        pltpu.sync_copy(zb, den_hbm.at[cid, pl.ds(off, ZR)])
        return c2

    lax.fori_loop(0, DRD // ZR, draind, 0)


def _k4(wm, ex, tgt_sc, tgt2):
    mesh = plsc.VectorSubcoreMesh(core_axis_name="c", subcore_axis_name="s")
    fn = pl.kernel(
        _k4_body,
        out_type=[
            jax.ShapeDtypeStruct((N, MSG), jnp.float32),
            jax.ShapeDtypeStruct((2, N2, 128), jnp.float32),
        ],
        mesh=mesh,
        scratch_types=[
            pltpu.VMEM_SHARED((N, MSG // 2), jnp.float32),
            pltpu.VMEM_SHARED((DND, 128), jnp.float32),
            pltpu.VMEM((ZR, 128), jnp.float32),
            pltpu.VMEM((ZR,), jnp.int32),
            pltpu.VMEM((SC,), jnp.int32),
            pltpu.VMEM((SC,), jnp.int32),
            pltpu.VMEM((SC, 128), jnp.float32),
        ],
    )
    return fn(wm, ex, tgt_sc, tgt2, jnp.arange(N, dtype=jnp.int32))


def _k5_body(x_ref, agg_ref, den_ref, wout_ref, bout_ref,
             wb1_ref, bb1_ref, wb2_ref, bb2_ref, ab_ref, out_ref):
    den = den_ref[...]
    hr = lax.broadcasted_iota(jnp.int32, (128, MSG), 0)
    cc = lax.broadcasted_iota(jnp.int32, (128, MSG), 1) // PH
    P16 = (hr == cc).astype(jnp.float32)
    dexp = jnp.dot(den, P16, preferred_element_type=jnp.float32) + SMALL
    agg = agg_ref[...] / dexp
    mp = jnp.dot(agg, wout_ref[...], preferred_element_type=jnp.float32) + bout_ref[...]
    x1 = x_ref[...] + ab_ref[0, 0] * mp
    hb = jnp.maximum(jnp.dot(x1, wb1_ref[...], preferred_element_type=jnp.float32)
                     + bb1_ref[...], 0.0)
    boom = jnp.dot(hb, wb2_ref[...], preferred_element_type=jnp.float32) + bb2_ref[...]
    out_ref[...] = x1 + ab_ref[0, 1] * boom


def _k5(x, agg, dens, Wout, bout, Wb1, bb1, Wb2, bb2, ab):
    return pl.pallas_call(
        _k5_body,
        grid=(N // BN,),
        in_specs=[
            pl.BlockSpec((BN, D), lambda i: (i, 0)),
            pl.BlockSpec((BN, MSG), lambda i: (i, 0)),
            pl.BlockSpec((BN, 128), lambda i: (i, 0)),
            pl.BlockSpec((MSG, D), lambda i: (0, 0)),
            pl.BlockSpec((1, D), lambda i: (0, 0)),
            pl.BlockSpec((D, INTER), lambda i: (0, 0)),
            pl.BlockSpec((1, INTER), lambda i: (0, 0)),
            pl.BlockSpec((INTER, D), lambda i: (0, 0)),
            pl.BlockSpec((1, D), lambda i: (0, 0)),
            pl.BlockSpec((1, 2), lambda i: (0, 0)),
        ],
        out_specs=pl.BlockSpec((BN, D), lambda i: (i, 0)),
        out_shape=jax.ShapeDtypeStruct((N, D), jnp.float32),
    )(x, agg, dens, Wout, bout, Wb1, bb1, Wb2, bb2, ab)


def kernel(x, adj_0, adj_1, adj_2, Wq_0, Wk_0, Wm1_0, bm1_0, Wm2_0, bm2_0,
           Wq_1, Wk_1, Wm1_1, bm1_1, Wm2_1, bm2_1,
           Wq_2, Wk_2, Wm1_2, bm1_2, Wm2_2, bm2_2,
           Wout, bout, Wb1, bb1, Wb2, bb2, alpha, beta):
    scale = PH ** (-0.5)
    adjs = [adj_0, adj_1, adj_2]
    Wqs = [Wq_0, Wq_1, Wq_2]
    Wks = [Wk_0, Wk_1, Wk_2]
    Wm1s = [Wm1_0, Wm1_1, Wm1_2]
    bm1s = [bm1_0, bm1_1, bm1_2]
    Wm2s = [Wm2_0, Wm2_1, Wm2_2]
    bm2s = [bm2_0, bm2_1, bm2_2]

    WS = jnp.stack([jnp.concatenate([Wm1s[t][:D], Wks[t]], axis=1) for t in range(ET)])
    WT = jnp.stack([jnp.concatenate([Wm1s[t][D:], Wqs[t] * scale], axis=1) for t in range(ET)])
    bS = jnp.stack([jnp.concatenate([bm1s[t], jnp.zeros((MSG,), jnp.float32)]) for t in range(ET)])[:, None, :]

    S, T = _k1(x, WS, WT, bS)
    Sflat = S.reshape(ET * N, TBL)
    Tflat = T.reshape(ET * N, TBL)

    pad = jnp.zeros((E2P - E2,), jnp.int32)
    src_all = jnp.concatenate(
        [jnp.concatenate([adjs[t][:, 0], adjs[t][:, 1], pad]) + t * N for t in range(ET)])
    tgt_sc = jnp.concatenate(
        [jnp.concatenate([adjs[t][:, 1], adjs[t][:, 0], pad]) for t in range(ET)])
    tgt_all = tgt_sc + jnp.repeat(jnp.arange(ET, dtype=jnp.int32) * N, E2P)

    hpre, qk = _k2(Sflat, Tflat, src_all, tgt_all)

    Wm2c = jnp.stack(Wm2s)
    bm2c = jnp.stack(bm2s)[:, None, :]
    wm, ex = _k3(hpre, qk, Wm2c, bm2c)

    tgt2 = jnp.stack([
        jnp.where(tgt_sc < N2, tgt_sc, N2),
        jnp.where(tgt_sc >= N2, tgt_sc - N2, N2),
    ])
    agg, den2 = _k4(wm, ex, tgt_sc, tgt2)
    dens = den2.reshape(N, 128)

    ab = jnp.stack([alpha, beta]).reshape(1, 2)
    return _k5(x, agg, dens, Wout, bout.reshape(1, D),
               Wb1, bb1.reshape(1, INTER), Wb2, bb2.reshape(1, D), ab)


# speedup vs baseline: 8.0053x; 1.2319x over previous
import functools

import jax
import jax.numpy as jnp
from jax import lax
from jax.experimental import pallas as pl
from jax.experimental.pallas import tpu as pltpu
from jax.experimental.pallas import tpu_sc as plsc

N = 10000; D = 256; H = 8; PH = 32; ET = 3; E = 53334; MSG = H * PH; INTER = 1024
SMALL = 1e-7
E2 = 2 * E
E2P = 108544
E3 = ET * E2P
TBL = 2 * D + MSG
NW = 32
EPW = E3 // NW
GC = 32
NGC = EPW // GC
NCG = E3 // GC
BE = 512
NBE = E3 // BE
NBT = E2P // BE
BN = 1000
SCK = 96
NSCK = E3 // 16 // SCK
ZR = 64
ZRD = 32
DR = 640
N2 = N // 2
DND = 5008
DRD = 320


def _k1_body(x_ref, ws_ref, wt_ref, bs_ref, s_ref, t_ref):
    xb = x_ref[...]
    s_ref[0] = jnp.dot(xb, ws_ref[0], preferred_element_type=jnp.float32) + bs_ref[0]
    t_ref[0] = jnp.dot(xb, wt_ref[0], preferred_element_type=jnp.float32)


def _k1(x, WS, WT, bS):
    return pl.pallas_call(
        _k1_body,
        grid=(ET, N // BN),
        in_specs=[
            pl.BlockSpec((BN, D), lambda t, i: (i, 0)),
            pl.BlockSpec((1, D, TBL), lambda t, i: (t, 0, 0)),
            pl.BlockSpec((1, D, TBL), lambda t, i: (t, 0, 0)),
            pl.BlockSpec((1, 1, TBL), lambda t, i: (t, 0, 0)),
        ],
        out_specs=[
            pl.BlockSpec((1, BN, TBL), lambda t, i: (t, i, 0)),
            pl.BlockSpec((1, BN, TBL), lambda t, i: (t, i, 0)),
        ],
        out_shape=[
            jax.ShapeDtypeStruct((ET, N, TBL), jnp.float32),
            jax.ShapeDtypeStruct((ET, N, TBL), jnp.float32),
        ],
    )(x, WS, WT, bS)


def _k2_body(s_hbm, t_hbm, idxcat_hbm, hpre_hbm, qk_hbm,
             idx0, idx1, srows0, srows1, trows0, trows1, hbuf, qbuf,
             sem_i0, sem_i1, sem_s0, sem_s1, sem_t0, sem_t1, sem_wh, sem_wq):
    cid = lax.axis_index("c")
    sid = lax.axis_index("s")
    wid = sid * 2 + cid
    base0 = wid * EPW
    gch0 = wid * NGC

    idxs = (idx0, idx1)
    srowss = (srows0, srows1)
    trowss = (trows0, trows1)
    sem_is = (sem_i0, sem_i1)
    sem_ss = (sem_s0, sem_s1)
    sem_ts = (sem_t0, sem_t1)

    def idx_load(c, b, sync=False):
        sl = pl.ds((gch0 + c) * (2 * GC), 2 * GC)
        if sync:
            pltpu.sync_copy(idxcat_hbm.at[sl], idxs[b])
        else:
            pltpu.async_copy(idxcat_hbm.at[sl], idxs[b], sem_is[b])

    def gather_issue(b):
        pltpu.async_copy(s_hbm.at[idxs[b].at[pl.ds(0, GC)]], srowss[b], sem_ss[b])
        pltpu.async_copy(t_hbm.at[idxs[b].at[pl.ds(GC, GC)]], trowss[b], sem_ts[b])

    def gather_wait(b):
        pltpu.make_async_copy(s_hbm.at[idxs[b].at[pl.ds(0, GC)]], srowss[b], sem_ss[b]).wait()
        pltpu.make_async_copy(t_hbm.at[idxs[b].at[pl.ds(GC, GC)]], trowss[b], sem_ts[b]).wait()

    def idx_wait(b):
        sl = pl.ds(0, 2 * GC)
        pltpu.make_async_copy(idxcat_hbm.at[sl], idxs[b], sem_is[b]).wait()

    def write_issue(row):
        pltpu.async_copy(hbuf, hpre_hbm.at[pl.ds(row, GC)], sem_wh)
        pltpu.async_copy(qbuf, qk_hbm.at[pl.ds(row, GC)], sem_wq)

    def write_wait():
        pltpu.make_async_copy(hbuf, hpre_hbm.at[pl.ds(0, GC)], sem_wh).wait()
        pltpu.make_async_copy(qbuf, qk_hbm.at[pl.ds(0, GC)], sem_wq).wait()

    idx_load(0, 0, sync=True)
    gather_issue(0)
    idx_load(1, 1)
    write_issue(E3)

    def pair(j, carry):
        for b in range(2):
            c = 2 * j + b
            gather_wait(b)
            idx_load(c + 2, b)
            write_wait()

            def edge(e4, c2):
                for u in range(4):
                    e = 4 * e4 + u
                    for i in range(2 * D // 16):
                        sl = pl.ds(16 * i, 16)
                        hbuf[e, sl] = srowss[b][e, sl] + trowss[b][e, sl]
                    for i in range(MSG // 16):
                        sl = pl.ds(2 * D + 16 * i, 16)
                        so = pl.ds(16 * i, 16)
                        qbuf[e, so] = srowss[b][e, sl] * trowss[b][e, sl]
                return c2

            lax.fori_loop(0, GC // 4, edge, 0)
            write_issue(base0 + c * GC)
            idx_wait(1 - b)
            gather_issue(1 - b)
        return carry

    lax.fori_loop(0, NGC // 2, pair, 0)

    gather_wait(0)
    idx_wait(1)
    write_wait()


def _k2(Sflat, Tflat, idxcat):
    mesh = plsc.VectorSubcoreMesh(core_axis_name="c", subcore_axis_name="s")
    fn = pl.kernel(
        _k2_body,
        out_type=[
            jax.ShapeDtypeStruct((E3 + GC, 2 * D), jnp.float32),
            jax.ShapeDtypeStruct((E3 + GC, MSG), jnp.float32),
        ],
        mesh=mesh,
        scratch_types=[
            pltpu.VMEM((2 * GC,), jnp.int32),
            pltpu.VMEM((2 * GC,), jnp.int32),
            pltpu.VMEM((GC, TBL), jnp.float32),
            pltpu.VMEM((GC, TBL), jnp.float32),
            pltpu.VMEM((GC, TBL), jnp.float32),
            pltpu.VMEM((GC, TBL), jnp.float32),
            pltpu.VMEM((GC, 2 * D), jnp.float32),
            pltpu.VMEM((GC, MSG), jnp.float32),
            pltpu.SemaphoreType.DMA,
            pltpu.SemaphoreType.DMA,
            pltpu.SemaphoreType.DMA,
            pltpu.SemaphoreType.DMA,
            pltpu.SemaphoreType.DMA,
            pltpu.SemaphoreType.DMA,
            pltpu.SemaphoreType.DMA,
            pltpu.SemaphoreType.DMA,
        ],
    )
    return fn(Sflat, Tflat, idxcat)


def _k3_body(hpre_ref, qk_ref, wm2_ref, bm2_ref, wm_ref, ex_ref):
    h = jnp.maximum(hpre_ref[...], 0.0)
    m = jnp.dot(h, wm2_ref[0], preferred_element_type=jnp.float32) + bm2_ref[0]
    m = jnp.maximum(m, 0.0)
    qk = qk_ref[...]
    ci = lax.broadcasted_iota(jnp.int32, (MSG, H), 0) // PH
    hj = lax.broadcasted_iota(jnp.int32, (MSG, H), 1)
    BD = (ci == hj).astype(jnp.float32)
    s = jnp.dot(qk, BD, preferred_element_type=jnp.float32)
    gid = pl.program_id(0)
    lrow = (gid % NBT) * BE + lax.broadcasted_iota(jnp.int32, (BE, H), 0)
    ex = jnp.where(lrow < E2, jnp.exp(s), 0.0)
    hr = lax.broadcasted_iota(jnp.int32, (H, MSG), 0)
    cc = lax.broadcasted_iota(jnp.int32, (H, MSG), 1) // PH
    P8 = (hr == cc).astype(jnp.float32)
    wm = m * jnp.dot(ex, P8, preferred_element_type=jnp.float32)
    wm_ref[...] = wm
    er = lax.broadcasted_iota(jnp.int32, (H, 128), 0)
    ec = lax.broadcasted_iota(jnp.int32, (H, 128), 1)
    ex_ref[...] = jnp.dot(ex, (er == ec).astype(jnp.float32),
                          preferred_element_type=jnp.float32)


def _k3(hpre, qk, Wm2s, bm2s):
    return pl.pallas_call(
        _k3_body,
        grid=(NBE,),
        in_specs=[
            pl.BlockSpec((BE, 2 * D), lambda i: (i, 0)),
            pl.BlockSpec((BE, MSG), lambda i: (i, 0)),
            pl.BlockSpec((1, 2 * D, MSG), lambda i: (i // NBT, 0, 0)),
            pl.BlockSpec((1, 1, MSG), lambda i: (i // NBT, 0, 0)),
        ],
        out_specs=[
            pl.BlockSpec((BE, MSG), lambda i: (i, 0)),
            pl.BlockSpec((BE, 128), lambda i: (i, 0)),
        ],
        out_shape=[
            jax.ShapeDtypeStruct((E3, MSG), jnp.float32),
            jax.ShapeDtypeStruct((E3, 128), jnp.float32),
        ],
    )(hpre, qk, Wm2s, bm2s)


def _k4a_body(wm_hbm, tgt_hbm, rows_hbm, agg_hbm, acc_agg, zb, ridx, tidx, wmbuf):
    cid = lax.axis_index("c")
    sid = lax.axis_index("s")
    ccol = cid * (MSG // 2)

    def zrow(r, c2):
        for i in range(8):
            zb[r, pl.ds(16 * i, 16)] = jnp.zeros((16,), jnp.float32)
        return c2

    lax.fori_loop(0, ZR, zrow, 0)

    base_rows = jnp.minimum(sid * DR, N - DR)

    def stripe_idx(off):
        pltpu.sync_copy(rows_hbm.at[pl.ds(off, ZR)], ridx)

    def zcopy(b, c2):
        off = base_rows + b * ZR
        stripe_idx(off)
        pltpu.sync_copy(zb, acc_agg.at[ridx])
        return c2

    lax.fori_loop(0, DR // ZR, zcopy, 0)
    plsc.subcore_barrier()

    def sc_chunk(j, c2):
        base = sid * (E3 // 16) + j * SCK
        pltpu.sync_copy(tgt_hbm.at[pl.ds(base, SCK)], tidx)
        pltpu.sync_copy(wm_hbm.at[pl.ds(base, SCK), pl.ds(ccol, MSG // 2)], wmbuf)
        pltpu.sync_copy(wmbuf, acc_agg.at[tidx], add=True)
        return c2

    lax.fori_loop(0, NSCK, sc_chunk, 0)
    plsc.subcore_barrier()

    def drain(b, c2):
        off = base_rows + b * ZR
        stripe_idx(off)
        pltpu.sync_copy(acc_agg.at[ridx], zb)
        pltpu.sync_copy(zb, agg_hbm.at[pl.ds(off, ZR), pl.ds(ccol, MSG // 2)])
        return c2

    lax.fori_loop(0, DR // ZR, drain, 0)


def _k4a(wm, tgt_sc):
    mesh = plsc.VectorSubcoreMesh(core_axis_name="c", subcore_axis_name="s")
    fn = pl.kernel(
        _k4a_body,
        out_type=[jax.ShapeDtypeStruct((N, MSG), jnp.float32)],
        mesh=mesh,
        scratch_types=[
            pltpu.VMEM_SHARED((N, MSG // 2), jnp.float32),
            pltpu.VMEM((ZR, 128), jnp.float32),
            pltpu.VMEM((ZR,), jnp.int32),
            pltpu.VMEM((SCK,), jnp.int32),
            pltpu.VMEM((SCK, 128), jnp.float32),
        ],
    )
    (agg,) = fn(wm, tgt_sc, jnp.arange(N, dtype=jnp.int32))
    return agg


def _k4b_body(ex_hbm, tgt2_hbm, rows_hbm, den_hbm, acc_den, zb, ridx, tidx2, exbuf):
    cid = lax.axis_index("c")
    sid = lax.axis_index("s")

    def zrow(r, c2):
        for i in range(8):
            zb[r, pl.ds(16 * i, 16)] = jnp.zeros((16,), jnp.float32)
        return c2

    lax.fori_loop(0, ZRD, zrow, 0)
    base_den = jnp.minimum(sid * DRD, DND - DRD)

    def stripe_idx(off):
        pltpu.sync_copy(rows_hbm.at[pl.ds(off, ZRD)], ridx)

    def zden(b, c2):
        off = base_den + b * ZRD
        stripe_idx(off)
        pltpu.sync_copy(zb, acc_den.at[ridx])
        return c2

    lax.fori_loop(0, DRD // ZRD, zden, 0)
    plsc.subcore_barrier()

    def den_chunk(j, c2):
        base = sid * (E3 // 16) + j * SCK
        pltpu.sync_copy(tgt2_hbm.at[pl.ds(cid * E3 + base, SCK)], tidx2)
        pltpu.sync_copy(ex_hbm.at[pl.ds(base, SCK)], exbuf)
        pltpu.sync_copy(exbuf, acc_den.at[tidx2], add=True)
        return c2

    lax.fori_loop(0, NSCK, den_chunk, 0)
    plsc.subcore_barrier()

    base_dd = jnp.minimum(sid * DRD, N2 - DRD)

    def draind(b, c2):
        off = base_dd + b * ZRD
        stripe_idx(off)
        pltpu.sync_copy(acc_den.at[ridx], zb)
        pltpu.sync_copy(zb, den_hbm.at[cid, pl.ds(off, ZRD)])
        return c2

    lax.fori_loop(0, DRD // ZRD, draind, 0)


def _k4b(ex, tgt2):
    mesh = plsc.VectorSubcoreMesh(core_axis_name="c", subcore_axis_name="s")
    fn = pl.kernel(
        _k4b_body,
        out_type=[jax.ShapeDtypeStruct((2, N2, 128), jnp.float32)],
        mesh=mesh,
        scratch_types=[
            pltpu.VMEM_SHARED((DND, 128), jnp.float32),
            pltpu.VMEM((ZRD, 128), jnp.float32),
            pltpu.VMEM((ZRD,), jnp.int32),
            pltpu.VMEM((SCK,), jnp.int32),
            pltpu.VMEM((SCK, 128), jnp.float32),
        ],
    )
    (den2,) = fn(ex, tgt2, jnp.arange(N, dtype=jnp.int32))
    return den2


def _k5_body(x_ref, agg_ref, den_ref, wout_ref, bout_ref,
             wb1_ref, bb1_ref, wb2_ref, bb2_ref, ab_ref, out_ref):
    den = den_ref[...]
    hr = lax.broadcasted_iota(jnp.int32, (128, MSG), 0)
    cc = lax.broadcasted_iota(jnp.int32, (128, MSG), 1) // PH
    P16 = (hr == cc).astype(jnp.float32)
    dexp = jnp.dot(den, P16, preferred_element_type=jnp.float32) + SMALL
    agg = agg_ref[...] / dexp
    mp = jnp.dot(agg, wout_ref[...], preferred_element_type=jnp.float32) + bout_ref[...]
    x1 = x_ref[...] + ab_ref[0, 0] * mp
    hb = jnp.maximum(jnp.dot(x1, wb1_ref[...], preferred_element_type=jnp.float32)
                     + bb1_ref[...], 0.0)
    boom = jnp.dot(hb, wb2_ref[...], preferred_element_type=jnp.float32) + bb2_ref[...]
    out_ref[...] = x1 + ab_ref[0, 1] * boom


def _k5(x, agg, dens, Wout, bout, Wb1, bb1, Wb2, bb2, ab):
    return pl.pallas_call(
        _k5_body,
        grid=(N // BN,),
        in_specs=[
            pl.BlockSpec((BN, D), lambda i: (i, 0)),
            pl.BlockSpec((BN, MSG), lambda i: (i, 0)),
            pl.BlockSpec((BN, 128), lambda i: (i, 0)),
            pl.BlockSpec((MSG, D), lambda i: (0, 0)),
            pl.BlockSpec((1, D), lambda i: (0, 0)),
            pl.BlockSpec((D, INTER), lambda i: (0, 0)),
            pl.BlockSpec((1, INTER), lambda i: (0, 0)),
            pl.BlockSpec((INTER, D), lambda i: (0, 0)),
            pl.BlockSpec((1, D), lambda i: (0, 0)),
            pl.BlockSpec((1, 2), lambda i: (0, 0)),
        ],
        out_specs=pl.BlockSpec((BN, D), lambda i: (i, 0)),
        out_shape=jax.ShapeDtypeStruct((N, D), jnp.float32),
    )(x, agg, dens, Wout, bout, Wb1, bb1, Wb2, bb2, ab)


def kernel(x, adj_0, adj_1, adj_2, Wq_0, Wk_0, Wm1_0, bm1_0, Wm2_0, bm2_0,
           Wq_1, Wk_1, Wm1_1, bm1_1, Wm2_1, bm2_1,
           Wq_2, Wk_2, Wm1_2, bm1_2, Wm2_2, bm2_2,
           Wout, bout, Wb1, bb1, Wb2, bb2, alpha, beta):
    scale = PH ** (-0.5)
    adjs = [adj_0, adj_1, adj_2]
    Wqs = [Wq_0, Wq_1, Wq_2]
    Wks = [Wk_0, Wk_1, Wk_2]
    Wm1s = [Wm1_0, Wm1_1, Wm1_2]
    bm1s = [bm1_0, bm1_1, bm1_2]
    Wm2s = [Wm2_0, Wm2_1, Wm2_2]
    bm2s = [bm2_0, bm2_1, bm2_2]

    WS = jnp.stack([jnp.concatenate([Wm1s[t][:D], Wks[t]], axis=1) for t in range(ET)])
    WT = jnp.stack([jnp.concatenate([Wm1s[t][D:], Wqs[t] * scale], axis=1) for t in range(ET)])
    bS = jnp.stack([jnp.concatenate([bm1s[t], jnp.zeros((MSG,), jnp.float32)]) for t in range(ET)])[:, None, :]

    S, T = _k1(x, WS, WT, bS)
    Sflat = S.reshape(ET * N, TBL)
    Tflat = T.reshape(ET * N, TBL)

    pad = jnp.zeros((E2P - E2,), jnp.int32)
    src_all = jnp.concatenate(
        [jnp.concatenate([adjs[t][:, 0], adjs[t][:, 1], pad]) + t * N for t in range(ET)])
    tgt_sc = jnp.concatenate(
        [jnp.concatenate([adjs[t][:, 1], adjs[t][:, 0], pad]) for t in range(ET)])
    tgt_all = tgt_sc + jnp.repeat(jnp.arange(ET, dtype=jnp.int32) * N, E2P)

    idxcat = jnp.concatenate([
        jnp.stack([src_all.reshape(NCG, GC), tgt_all.reshape(NCG, GC)],
                  axis=1).reshape(-1),
        jnp.zeros((4 * GC,), jnp.int32),
    ])
    hpre, qk = _k2(Sflat, Tflat, idxcat)

    Wm2c = jnp.stack(Wm2s)
    bm2c = jnp.stack(bm2s)[:, None, :]
    wm, ex = _k3(hpre, qk, Wm2c, bm2c)

    tgt2 = jnp.concatenate([
        jnp.where(tgt_sc < N2, tgt_sc, N2),
        jnp.where(tgt_sc >= N2, tgt_sc - N2, N2),
    ])
    agg = _k4a(wm, tgt_sc)
    den2 = _k4b(ex, tgt2)
    dens = den2.reshape(N, 128)

    ab = jnp.stack([alpha, beta]).reshape(1, 2)
    return _k5(x, agg, dens, Wout, bout.reshape(1, D),
               Wb1, bb1.reshape(1, INTER), Wb2, bb2.reshape(1, D), ab)
